# Initial kernel scaffold; baseline (speedup 1.0000x reference)
import jax
import jax.numpy as jnp
from jax.experimental import pallas as pl


def kernel(x, edge_index, W1, b1, g1, be1, pw, W2, b2, g2, be2):
    raise NotImplementedError("write your pallas kernel here")


# speedup vs baseline: 1.7021x; 1.7021x over previous
import functools

import jax
import jax.numpy as jnp
from jax import lax
from jax.experimental import pallas as pl
from jax.experimental.pallas import tpu as pltpu
from jax.experimental.pallas import tpu_sc as plsc

_D = 128
_NC = 2
_NS = 16
_NW = _NC * _NS
_CH = 128
_NPH = 2
_DEGW = 16


def _mesh():
    return plsc.VectorSubcoreMesh(core_axis_name="c", subcore_axis_name="s")


def _deg_partials(dst2, npad, steps):
    rows_per_tile = npad // _NS

    @functools.partial(
        pl.kernel,
        out_type=jax.ShapeDtypeStruct((_NC * npad,), jnp.float32),
        mesh=_mesh(),
        scratch_types=[
            pltpu.VMEM_SHARED((npad,), jnp.float32),
            pltpu.VMEM((steps, _CH), jnp.int32),
            pltpu.VMEM((_CH,), jnp.float32),
            pltpu.VMEM((rows_per_tile,), jnp.float32),
        ],
    )
    def k(dst_hbm, out_hbm, acc, dbuf, ones, stage):
        c = lax.axis_index("c")
        s = lax.axis_index("s")
        w = c * _NS + s

        @pl.loop(0, _CH, step=16)
        def _(i):
            ones[pl.ds(i, 16)] = jnp.ones((16,), jnp.float32)

        @pl.loop(0, rows_per_tile, step=16)
        def _(i):
            stage[pl.ds(i, 16)] = jnp.zeros((16,), jnp.float32)

        pltpu.sync_copy(stage, acc.at[pl.ds(s * rows_per_tile, rows_per_tile)])
        plsc.subcore_barrier()

        pltpu.sync_copy(dst_hbm.at[pl.ds(w * steps, steps)], dbuf)

        @pl.loop(0, steps)
        def _(g):
            pltpu.sync_copy(ones, acc.at[dbuf.at[g]], add=True)

        plsc.subcore_barrier()
        pltpu.sync_copy(acc.at[pl.ds(s * rows_per_tile, rows_per_tile)], stage)
        pltpu.sync_copy(
            stage, out_hbm.at[pl.ds(c * npad + s * rows_per_tile, rows_per_tile)])

    return k(dst2)


def _edge_partials(hws, src2, dst2, npad, steps):
    rows_per_tile = npad // _NS
    n_chunks = rows_per_tile // _CH
    ps = steps // _NPH

    @functools.partial(
        pl.kernel,
        out_type=jax.ShapeDtypeStruct((_NC, npad, _D), jnp.float32),
        mesh=_mesh(),
        scratch_types=[
            pltpu.VMEM_SHARED((npad, _D), jnp.float32),
            pltpu.VMEM((ps, _CH), jnp.int32),
            pltpu.VMEM((ps, _CH), jnp.int32),
            pltpu.VMEM((2, _CH, _D), jnp.float32),
            pltpu.SemaphoreType.DMA,
            pltpu.SemaphoreType.DMA,
        ],
    )
    def k(hws_hbm, src_hbm, dst_hbm, out_hbm, acc, sbuf, dbuf, rows, sem0, sem1):
        c = lax.axis_index("c")
        s = lax.axis_index("s")
        w = c * _NS + s
        sems = (sem0, sem1)

        @pl.loop(0, _CH)
        def _(i):
            @pl.loop(0, _D, step=16)
            def _(j):
                rows[0, i, pl.ds(j, 16)] = jnp.zeros((16,), jnp.float32)

        @pl.loop(0, n_chunks)
        def _(kk):
            pltpu.sync_copy(
                rows.at[0], acc.at[pl.ds(s * rows_per_tile + kk * _CH, _CH)])

        plsc.subcore_barrier()

        for ph in range(_NPH):
            base = w * steps + ph * ps
            pltpu.sync_copy(src_hbm.at[pl.ds(base, ps)], sbuf)
            pltpu.sync_copy(dst_hbm.at[pl.ds(base, ps)], dbuf)

            pltpu.async_copy(hws_hbm.at[sbuf.at[0]], rows.at[0], sem0)

            @pl.loop(0, ps, step=2)
            def _(g):
                for b in (0, 1):
                    gg = g + b

                    @pl.when(gg + 1 < ps)
                    def _():
                        pltpu.async_copy(
                            hws_hbm.at[sbuf.at[gg + 1]], rows.at[1 - b],
                            sems[1 - b])

                    pltpu.make_async_copy(
                        hws_hbm.at[sbuf.at[gg]], rows.at[b], sems[b]).wait()
                    pltpu.sync_copy(rows.at[b], acc.at[dbuf.at[gg]], add=True)

        plsc.subcore_barrier()

        @pl.loop(0, n_chunks)
        def _(kk):
            r0 = s * rows_per_tile + kk * _CH
            pltpu.sync_copy(acc.at[pl.ds(r0, _CH)], rows.at[0])
            pltpu.sync_copy(rows.at[0], out_hbm.at[c, pl.ds(r0, _CH)])

    return k(hws, src2, dst2)


_R = 2000


def _row_spec(r, d):
    return pl.BlockSpec((r, d), lambda i: (i, 0))


def _full_spec(shape):
    nd = len(shape)
    return pl.BlockSpec(shape, lambda i: (0,) * nd)


def _mm(x, w):
    n, d = x.shape

    def body(x_ref, w_ref, o_ref):
        o_ref[...] = jnp.dot(x_ref[...], w_ref[...],
                             preferred_element_type=jnp.float32)

    return pl.pallas_call(
        body,
        grid=(n // _R,),
        in_specs=[_row_spec(_R, d), _full_spec(w.shape)],
        out_specs=_row_spec(_R, d),
        out_shape=jax.ShapeDtypeStruct((n, d), jnp.float32),
    )(x, w)


def _scale(hw, d0, d1):
    n, d = hw.shape

    def body(hw_ref, d0_ref, d1_ref, o_ref):
        dinv = lax.rsqrt(d0_ref[...] + d1_ref[...] + 1.0)
        o_ref[...] = hw_ref[...] * dinv

    return pl.pallas_call(
        body,
        grid=(n // _R,),
        in_specs=[_row_spec(_R, d), _row_spec(_R, 1), _row_spec(_R, 1)],
        out_specs=_row_spec(_R, d),
        out_shape=jax.ShapeDtypeStruct((n, d), jnp.float32),
    )(hw, d0, d1)


def _norm_act(p0, p1, hws, d0, d1, b, g, be, pw):
    dinv = lax.rsqrt(d0 + d1 + 1.0)
    agg = dinv * (p0 + p1 + hws) + b
    mu = jnp.mean(agg, axis=-1, keepdims=True)
    var = jnp.mean((agg - mu) ** 2, axis=-1, keepdims=True)
    h = (agg - mu) * lax.rsqrt(var + 1e-5) * g + be
    return jnp.where(h > 0, h, h * pw)


def _mid(p0, p1, hws, d0, d1, b, g, be, pw, w2):
    n, d = hws.shape

    def body(p0_r, p1_r, hws_r, d0_r, d1_r, b_r, g_r, be_r, pw_r, w2_r, o_ref):
        h = _norm_act(p0_r[...], p1_r[...], hws_r[...], d0_r[...], d1_r[...],
                      b_r[...], g_r[...], be_r[...], pw_r[...])
        dinv = lax.rsqrt(d0_r[...] + d1_r[...] + 1.0)
        o_ref[...] = jnp.dot(h, w2_r[...],
                             preferred_element_type=jnp.float32) * dinv

    return pl.pallas_call(
        body,
        grid=(n // _R,),
        in_specs=[_row_spec(_R, d), _row_spec(_R, d), _row_spec(_R, d),
                  _row_spec(_R, 1), _row_spec(_R, 1),
                  _full_spec((1, d)), _full_spec((1, d)), _full_spec((1, d)),
                  _full_spec((1, d)), _full_spec((d, d))],
        out_specs=_row_spec(_R, d),
        out_shape=jax.ShapeDtypeStruct((n, d), jnp.float32),
    )(p0, p1, hws, d0, d1, b, g, be, pw, w2)


def _post(p0, p1, hws, d0, d1, b, g, be, pw):
    n, d = hws.shape

    def body(p0_r, p1_r, hws_r, d0_r, d1_r, b_r, g_r, be_r, pw_r, o_ref):
        o_ref[...] = _norm_act(p0_r[...], p1_r[...], hws_r[...], d0_r[...],
                               d1_r[...], b_r[...], g_r[...], be_r[...],
                               pw_r[...])

    return pl.pallas_call(
        body,
        grid=(n // _R,),
        in_specs=[_row_spec(_R, d), _row_spec(_R, d), _row_spec(_R, d),
                  _row_spec(_R, 1), _row_spec(_R, 1),
                  _full_spec((1, d)), _full_spec((1, d)), _full_spec((1, d)),
                  _full_spec((1, d))],
        out_specs=_row_spec(_R, d),
        out_shape=jax.ShapeDtypeStruct((n, d), jnp.float32),
    )(p0, p1, hws, d0, d1, b, g, be, pw)


def kernel(x, edge_index, W1, b1, g1, be1, pw, W2, b2, g2, be2):
    n, d = x.shape
    e = edge_index.shape[1]

    align = 2 * _NPH * 8
    steps = -(-e // (_NW * _CH * align)) * align
    epad = _NW * steps * _CH
    npad = -(-(n + 1) // (_NS * _CH)) * (_NS * _CH)

    src = edge_index[0]
    dst = edge_index[1]
    padn = epad - e
    src2 = jnp.concatenate([src, jnp.zeros((padn,), jnp.int32)]).reshape(-1, _CH)
    dst2 = jnp.concatenate([dst, jnp.full((padn,), n, jnp.int32)]).reshape(-1, _CH)

    degp = _deg_partials(dst2, npad, steps).reshape(_NC, npad)
    hw1 = _mm(x, W1)
    d0 = degp[0, :n, None]
    d1 = degp[1, :n, None]

    hws1 = _scale(hw1, d0, d1)
    P1 = _edge_partials(hws1, src2, dst2, npad, steps)
    b1r, g1r, be1r, pwr = (v.reshape(1, d) for v in (b1, g1, be1, pw))
    hws2 = _mid(P1[0, :n], P1[1, :n], hws1, d0, d1, b1r, g1r, be1r, pwr, W2)

    P2 = _edge_partials(hws2, src2, dst2, npad, steps)
    b2r, g2r, be2r = (v.reshape(1, d) for v in (b2, g2, be2))
    return _post(P2[0, :n], P2[1, :n], hws2, d0, d1, b2r, g2r, be2r, pwr)


# speedup vs baseline: 10.2768x; 6.0376x over previous
import functools

import jax
import jax.numpy as jnp
from jax import lax
from jax.experimental import pallas as pl
from jax.experimental.pallas import tpu as pltpu
from jax.experimental.pallas import tpu_sc as plsc

_D = 128
_NC = 2
_NS = 16
_NW = _NC * _NS
_CH = 128
_NPH = 2
_DEGW = 16


def _mesh():
    return plsc.VectorSubcoreMesh(core_axis_name="c", subcore_axis_name="s")


def _deg_partials(dst2, npad, steps):
    rows_per_tile = npad // _NS

    @functools.partial(
        pl.kernel,
        out_type=jax.ShapeDtypeStruct((_NC * npad,), jnp.float32),
        mesh=_mesh(),
        scratch_types=[
            pltpu.VMEM_SHARED((npad,), jnp.float32),
            pltpu.VMEM((steps, _CH), jnp.int32),
            pltpu.VMEM((_CH,), jnp.float32),
            pltpu.VMEM((rows_per_tile,), jnp.float32),
        ],
    )
    def k(dst_hbm, out_hbm, acc, dbuf, ones, stage):
        c = lax.axis_index("c")
        s = lax.axis_index("s")
        w = c * _NS + s

        @pl.loop(0, _CH, step=16)
        def _(i):
            ones[pl.ds(i, 16)] = jnp.ones((16,), jnp.float32)

        @pl.loop(0, rows_per_tile, step=16)
        def _(i):
            stage[pl.ds(i, 16)] = jnp.zeros((16,), jnp.float32)

        pltpu.sync_copy(stage, acc.at[pl.ds(s * rows_per_tile, rows_per_tile)])
        plsc.subcore_barrier()

        pltpu.sync_copy(dst_hbm.at[pl.ds(w * steps, steps)], dbuf)

        @pl.loop(0, steps)
        def _(g):
            pltpu.sync_copy(ones, acc.at[dbuf.at[g]], add=True)

        plsc.subcore_barrier()
        pltpu.sync_copy(acc.at[pl.ds(s * rows_per_tile, rows_per_tile)], stage)
        pltpu.sync_copy(
            stage, out_hbm.at[pl.ds(c * npad + s * rows_per_tile, rows_per_tile)])

    return k(dst2)


def _edge_partials(hws, src2, dst2, npad, steps):
    rows_per_tile = npad // _NS
    n_chunks = rows_per_tile // _CH
    ps = steps // _NPH

    @functools.partial(
        pl.kernel,
        out_type=jax.ShapeDtypeStruct((_NC, npad, _D), jnp.float32),
        mesh=_mesh(),
        scratch_types=[
            pltpu.VMEM_SHARED((npad, _D), jnp.float32),
            pltpu.VMEM((ps, _CH), jnp.int32),
            pltpu.VMEM((ps, _CH), jnp.int32),
            pltpu.VMEM((2, _CH, _D), jnp.float32),
            pltpu.SemaphoreType.DMA,
            pltpu.SemaphoreType.DMA,
        ],
    )
    def k(hws_hbm, src_hbm, dst_hbm, out_hbm, acc, sbuf, dbuf, rows, sem0, sem1):
        c = lax.axis_index("c")
        s = lax.axis_index("s")
        w = c * _NS + s
        sems = (sem0, sem1)

        @pl.loop(0, _CH)
        def _(i):
            @pl.loop(0, _D, step=16)
            def _(j):
                rows[0, i, pl.ds(j, 16)] = jnp.zeros((16,), jnp.float32)

        @pl.loop(0, n_chunks)
        def _(kk):
            pltpu.sync_copy(
                rows.at[0], acc.at[pl.ds(s * rows_per_tile + kk * _CH, _CH)])

        plsc.subcore_barrier()

        for ph in range(_NPH):
            base = w * steps + ph * ps
            pltpu.sync_copy(src_hbm.at[pl.ds(base, ps)], sbuf)
            pltpu.sync_copy(dst_hbm.at[pl.ds(base, ps)], dbuf)

            pltpu.async_copy(hws_hbm.at[sbuf.at[0]], rows.at[0], sem0)

            @pl.loop(0, ps, step=2)
            def _(g):
                for b in (0, 1):
                    gg = g + b

                    @pl.when(gg + 1 < ps)
                    def _():
                        pltpu.async_copy(
                            hws_hbm.at[sbuf.at[gg + 1]], rows.at[1 - b],
                            sems[1 - b])

                    pltpu.make_async_copy(
                        hws_hbm.at[sbuf.at[gg]], rows.at[b], sems[b]).wait()
                    pltpu.sync_copy(rows.at[b], acc.at[dbuf.at[gg]], add=True)

        plsc.subcore_barrier()

        @pl.loop(0, n_chunks)
        def _(kk):
            r0 = s * rows_per_tile + kk * _CH
            pltpu.sync_copy(acc.at[pl.ds(r0, _CH)], rows.at[0])
            pltpu.sync_copy(rows.at[0], out_hbm.at[c, pl.ds(r0, _CH)])

    return k(hws, src2, dst2)


_R = 2000


def _row_spec(r, d):
    return pl.BlockSpec((r, d), lambda i: (i, 0))


def _full_spec(shape):
    nd = len(shape)
    return pl.BlockSpec(shape, lambda i: (0,) * nd)


def _mm(x, w):
    n, d = x.shape

    def body(x_ref, w_ref, o_ref):
        o_ref[...] = jnp.dot(x_ref[...], w_ref[...],
                             preferred_element_type=jnp.float32)

    return pl.pallas_call(
        body,
        grid=(n // _R,),
        in_specs=[_row_spec(_R, d), _full_spec(w.shape)],
        out_specs=_row_spec(_R, d),
        out_shape=jax.ShapeDtypeStruct((n, d), jnp.float32),
    )(x, w)


def _scale(hw, d0, d1):
    n, d = hw.shape

    def body(hw_ref, d0_ref, d1_ref, o_ref):
        dinv = lax.rsqrt(d0_ref[...] + d1_ref[...] + 1.0)
        o_ref[...] = hw_ref[...] * dinv

    return pl.pallas_call(
        body,
        grid=(n // _R,),
        in_specs=[_row_spec(_R, d), _row_spec(_R, 1), _row_spec(_R, 1)],
        out_specs=_row_spec(_R, d),
        out_shape=jax.ShapeDtypeStruct((n, d), jnp.float32),
    )(hw, d0, d1)


def _norm_act(p0, p1, hws, d0, d1, b, g, be, pw):
    dinv = lax.rsqrt(d0 + d1 + 1.0)
    agg = dinv * (p0 + p1 + hws) + b
    mu = jnp.mean(agg, axis=-1, keepdims=True)
    var = jnp.mean((agg - mu) ** 2, axis=-1, keepdims=True)
    h = (agg - mu) * lax.rsqrt(var + 1e-5) * g + be
    return jnp.where(h > 0, h, h * pw)


def _mid(p0, p1, hws, d0, d1, b, g, be, pw, w2):
    n, d = hws.shape

    def body(p0_r, p1_r, hws_r, d0_r, d1_r, b_r, g_r, be_r, pw_r, w2_r, o_ref):
        h = _norm_act(p0_r[...], p1_r[...], hws_r[...], d0_r[...], d1_r[...],
                      b_r[...], g_r[...], be_r[...], pw_r[...])
        dinv = lax.rsqrt(d0_r[...] + d1_r[...] + 1.0)
        o_ref[...] = jnp.dot(h, w2_r[...],
                             preferred_element_type=jnp.float32) * dinv

    return pl.pallas_call(
        body,
        grid=(n // _R,),
        in_specs=[_row_spec(_R, d), _row_spec(_R, d), _row_spec(_R, d),
                  _row_spec(_R, 1), _row_spec(_R, 1),
                  _full_spec((1, d)), _full_spec((1, d)), _full_spec((1, d)),
                  _full_spec((1, d)), _full_spec((d, d))],
        out_specs=_row_spec(_R, d),
        out_shape=jax.ShapeDtypeStruct((n, d), jnp.float32),
    )(p0, p1, hws, d0, d1, b, g, be, pw, w2)


def _post(p0, p1, hws, d0, d1, b, g, be, pw):
    n, d = hws.shape

    def body(p0_r, p1_r, hws_r, d0_r, d1_r, b_r, g_r, be_r, pw_r, o_ref):
        o_ref[...] = _norm_act(p0_r[...], p1_r[...], hws_r[...], d0_r[...],
                               d1_r[...], b_r[...], g_r[...], be_r[...],
                               pw_r[...])

    return pl.pallas_call(
        body,
        grid=(n // _R,),
        in_specs=[_row_spec(_R, d), _row_spec(_R, d), _row_spec(_R, d),
                  _row_spec(_R, 1), _row_spec(_R, 1),
                  _full_spec((1, d)), _full_spec((1, d)), _full_spec((1, d)),
                  _full_spec((1, d))],
        out_specs=_row_spec(_R, d),
        out_shape=jax.ShapeDtypeStruct((n, d), jnp.float32),
    )(p0, p1, hws, d0, d1, b, g, be, pw)


def kernel(x, edge_index, W1, b1, g1, be1, pw, W2, b2, g2, be2):
    n, d = x.shape
    e = edge_index.shape[1]

    align = _NPH * 8
    steps = -(-e // (_NW * _CH * align)) * align
    epad = _NW * steps * _CH
    npad = -(-(n + 1) // (_NS * _CH)) * (_NS * _CH)

    src = edge_index[0]
    dst = edge_index[1]
    padn = epad - e
    pad_dst = n + jnp.arange(padn, dtype=jnp.int32) % (npad - n)
    src2 = jnp.concatenate([src, jnp.zeros((padn,), jnp.int32)]).reshape(-1, _CH)
    dst2 = jnp.concatenate([dst, pad_dst]).reshape(-1, _CH)

    degp = _deg_partials(dst2, npad, steps).reshape(_NC, npad)
    hw1 = _mm(x, W1)
    d0 = degp[0, :n, None]
    d1 = degp[1, :n, None]

    hws1 = _scale(hw1, d0, d1)
    P1 = _edge_partials(hws1, src2, dst2, npad, steps)
    b1r, g1r, be1r, pwr = (v.reshape(1, d) for v in (b1, g1, be1, pw))
    hws2 = _mid(P1[0, :n], P1[1, :n], hws1, d0, d1, b1r, g1r, be1r, pwr, W2)

    P2 = _edge_partials(hws2, src2, dst2, npad, steps)
    b2r, g2r, be2r = (v.reshape(1, d) for v in (b2, g2, be2))
    return _post(P2[0, :n], P2[1, :n], hws2, d0, d1, b2r, g2r, be2r, pwr)


# speedup vs baseline: 31.4848x; 3.0637x over previous
import functools

import jax
import jax.numpy as jnp
from jax import lax
from jax.experimental import pallas as pl
from jax.experimental.pallas import tpu as pltpu
from jax.experimental.pallas import tpu_sc as plsc

_D = 128
_NC = 2
_NS = 16
_NW = _NC * _NS
_CH = 128
_NPH = 2
_DEGW = 16


def _mesh():
    return plsc.VectorSubcoreMesh(core_axis_name="c", subcore_axis_name="s")


def _deg_partials(dst2, npad, steps):
    rows_per_tile = npad // _NS

    @functools.partial(
        pl.kernel,
        out_type=jax.ShapeDtypeStruct((_NC * npad,), jnp.float32),
        mesh=_mesh(),
        scratch_types=[
            pltpu.VMEM_SHARED((npad,), jnp.float32),
            pltpu.VMEM((steps, _CH), jnp.int32),
            pltpu.VMEM((_CH,), jnp.float32),
            pltpu.VMEM((rows_per_tile,), jnp.float32),
        ],
    )
    def k(dst_hbm, out_hbm, acc, dbuf, ones, stage):
        c = lax.axis_index("c")
        s = lax.axis_index("s")
        w = c * _NS + s

        @pl.loop(0, _CH, step=16)
        def _(i):
            ones[pl.ds(i, 16)] = jnp.ones((16,), jnp.float32)

        @pl.loop(0, rows_per_tile, step=16)
        def _(i):
            stage[pl.ds(i, 16)] = jnp.zeros((16,), jnp.float32)

        pltpu.sync_copy(stage, acc.at[pl.ds(s * rows_per_tile, rows_per_tile)])
        plsc.subcore_barrier()

        pltpu.sync_copy(dst_hbm.at[pl.ds(w * steps, steps)], dbuf)

        @pl.loop(0, steps)
        def _(g):
            pltpu.sync_copy(ones, acc.at[dbuf.at[g]], add=True)

        plsc.subcore_barrier()
        pltpu.sync_copy(acc.at[pl.ds(s * rows_per_tile, rows_per_tile)], stage)
        pltpu.sync_copy(
            stage, out_hbm.at[pl.ds(c * npad + s * rows_per_tile, rows_per_tile)])

    return k(dst2)


def _edge_partials(hws, src2, dst2, npad, steps):
    rows_per_tile = npad // _NS
    n_chunks = rows_per_tile // _CH
    ps = steps // _NPH

    @functools.partial(
        pl.kernel,
        out_type=jax.ShapeDtypeStruct((_NC, npad, _D), jnp.float32),
        mesh=_mesh(),
        scratch_types=[
            pltpu.VMEM_SHARED((npad, _D), jnp.float32),
            pltpu.VMEM((ps, _CH), jnp.int32),
            pltpu.VMEM((ps, _CH), jnp.int32),
            pltpu.VMEM((2, _CH, _D), jnp.float32),
            pltpu.SemaphoreType.DMA,
            pltpu.SemaphoreType.DMA,
        ],
    )
    def k(hws_hbm, src_hbm, dst_hbm, out_hbm, acc, sbuf, dbuf, rows, sem0, sem1):
        c = lax.axis_index("c")
        s = lax.axis_index("s")
        w = c * _NS + s
        sems = (sem0, sem1)

        @pl.loop(0, _CH)
        def _(i):
            @pl.loop(0, _D, step=16)
            def _(j):
                rows[0, i, pl.ds(j, 16)] = jnp.zeros((16,), jnp.float32)

        @pl.loop(0, n_chunks)
        def _(kk):
            pltpu.sync_copy(
                rows.at[0], acc.at[pl.ds(s * rows_per_tile + kk * _CH, _CH)])

        plsc.subcore_barrier()

        for ph in range(_NPH):
            base = w * steps + ph * ps
            pltpu.sync_copy(src_hbm.at[pl.ds(base, ps)], sbuf)
            pltpu.sync_copy(dst_hbm.at[pl.ds(base, ps)], dbuf)

            pltpu.async_copy(hws_hbm.at[sbuf.at[0]], rows.at[0], sem0)

            @pl.loop(0, ps, step=2)
            def _(g):
                for b in (0, 1):
                    gg = g + b

                    @pl.when(gg + 1 < ps)
                    def _():
                        pltpu.async_copy(
                            hws_hbm.at[sbuf.at[gg + 1]], rows.at[1 - b],
                            sems[1 - b])

                    pltpu.make_async_copy(
                        hws_hbm.at[sbuf.at[gg]], rows.at[b], sems[b]).wait()
                    pltpu.sync_copy(rows.at[b], acc.at[dbuf.at[gg]], add=True)

        plsc.subcore_barrier()

        @pl.loop(0, n_chunks)
        def _(kk):
            r0 = s * rows_per_tile + kk * _CH
            pltpu.sync_copy(acc.at[pl.ds(r0, _CH)], rows.at[0])
            pltpu.sync_copy(rows.at[0], out_hbm.at[c, pl.ds(r0, _CH)])

    return k(hws, src2, dst2)


_R = 2000


def _row_spec(r, d):
    return pl.BlockSpec((r, d), lambda i: (i, 0))


def _full_spec(shape):
    nd = len(shape)
    return pl.BlockSpec(shape, lambda i: (0,) * nd)


def _mm(x, w):
    n, d = x.shape

    def body(x_ref, w_ref, o_ref):
        o_ref[...] = jnp.dot(x_ref[...], w_ref[...],
                             preferred_element_type=jnp.float32)

    return pl.pallas_call(
        body,
        grid=(n // _R,),
        in_specs=[_row_spec(_R, d), _full_spec(w.shape)],
        out_specs=_row_spec(_R, d),
        out_shape=jax.ShapeDtypeStruct((n, d), jnp.float32),
    )(x, w)


def _scale(hw, d0, d1):
    n, d = hw.shape

    def body(hw_ref, d0_ref, d1_ref, o_ref):
        dinv = lax.rsqrt(d0_ref[...] + d1_ref[...] + 1.0)
        o_ref[...] = hw_ref[...] * dinv

    return pl.pallas_call(
        body,
        grid=(n // _R,),
        in_specs=[_row_spec(_R, d), _row_spec(_R, 1), _row_spec(_R, 1)],
        out_specs=_row_spec(_R, d),
        out_shape=jax.ShapeDtypeStruct((n, d), jnp.float32),
    )(hw, d0, d1)


def _norm_act(p0, p1, hws, d0, d1, b, g, be, pw):
    dinv = lax.rsqrt(d0 + d1 + 1.0)
    agg = dinv * (p0 + p1 + hws) + b
    mu = jnp.mean(agg, axis=-1, keepdims=True)
    var = jnp.mean((agg - mu) ** 2, axis=-1, keepdims=True)
    h = (agg - mu) * lax.rsqrt(var + 1e-5) * g + be
    return jnp.where(h > 0, h, h * pw)


def _mid(p0, p1, hws, d0, d1, b, g, be, pw, w2):
    n, d = hws.shape

    def body(p0_r, p1_r, hws_r, d0_r, d1_r, b_r, g_r, be_r, pw_r, w2_r, o_ref):
        h = _norm_act(p0_r[...], p1_r[...], hws_r[...], d0_r[...], d1_r[...],
                      b_r[...], g_r[...], be_r[...], pw_r[...])
        dinv = lax.rsqrt(d0_r[...] + d1_r[...] + 1.0)
        o_ref[...] = jnp.dot(h, w2_r[...],
                             preferred_element_type=jnp.float32) * dinv

    return pl.pallas_call(
        body,
        grid=(n // _R,),
        in_specs=[_row_spec(_R, d), _row_spec(_R, d), _row_spec(_R, d),
                  _row_spec(_R, 1), _row_spec(_R, 1),
                  _full_spec((1, d)), _full_spec((1, d)), _full_spec((1, d)),
                  _full_spec((1, d)), _full_spec((d, d))],
        out_specs=_row_spec(_R, d),
        out_shape=jax.ShapeDtypeStruct((n, d), jnp.float32),
    )(p0, p1, hws, d0, d1, b, g, be, pw, w2)


def _post(p0, p1, hws, d0, d1, b, g, be, pw):
    n, d = hws.shape

    def body(p0_r, p1_r, hws_r, d0_r, d1_r, b_r, g_r, be_r, pw_r, o_ref):
        o_ref[...] = _norm_act(p0_r[...], p1_r[...], hws_r[...], d0_r[...],
                               d1_r[...], b_r[...], g_r[...], be_r[...],
                               pw_r[...])

    return pl.pallas_call(
        body,
        grid=(n // _R,),
        in_specs=[_row_spec(_R, d), _row_spec(_R, d), _row_spec(_R, d),
                  _row_spec(_R, 1), _row_spec(_R, 1),
                  _full_spec((1, d)), _full_spec((1, d)), _full_spec((1, d)),
                  _full_spec((1, d))],
        out_specs=_row_spec(_R, d),
        out_shape=jax.ShapeDtypeStruct((n, d), jnp.float32),
    )(p0, p1, hws, d0, d1, b, g, be, pw)


def kernel(x, edge_index, W1, b1, g1, be1, pw, W2, b2, g2, be2):
    n, d = x.shape
    e = edge_index.shape[1]

    align = _NPH * 8
    steps = -(-e // (_NW * _CH * align)) * align
    epad = _NW * steps * _CH
    npad = -(-(n + 1) // (_NS * _CH)) * (_NS * _CH)

    src = edge_index[0]
    dst = edge_index[1]
    padn = epad - e
    pad_iota = jnp.arange(padn, dtype=jnp.int32)
    pad_dst = n + pad_iota % (npad - n)
    pad_src = pad_iota % n
    src2 = jnp.concatenate([src, pad_src]).reshape(-1, _CH)
    dst2 = jnp.concatenate([dst, pad_dst]).reshape(-1, _CH)

    degp = _deg_partials(dst2, npad, steps).reshape(_NC, npad)
    hw1 = _mm(x, W1)
    d0 = degp[0, :n, None]
    d1 = degp[1, :n, None]

    hws1 = _scale(hw1, d0, d1)
    P1 = _edge_partials(hws1, src2, dst2, npad, steps)
    b1r, g1r, be1r, pwr = (v.reshape(1, d) for v in (b1, g1, be1, pw))
    hws2 = _mid(P1[0, :n], P1[1, :n], hws1, d0, d1, b1r, g1r, be1r, pwr, W2)

    P2 = _edge_partials(hws2, src2, dst2, npad, steps)
    b2r, g2r, be2r = (v.reshape(1, d) for v in (b2, g2, be2))
    return _post(P2[0, :n], P2[1, :n], hws2, d0, d1, b2r, g2r, be2r, pwr)


# speedup vs baseline: 33.0186x; 1.0487x over previous
import functools

import jax
import jax.numpy as jnp
from jax import lax
from jax.experimental import pallas as pl
from jax.experimental.pallas import tpu as pltpu
from jax.experimental.pallas import tpu_sc as plsc

_D = 128
_NC = 2
_NS = 16
_NW = _NC * _NS
_CH = 128
_NPH = 2
_DEGW = 16


def _mesh():
    return plsc.VectorSubcoreMesh(core_axis_name="c", subcore_axis_name="s")


def _deg_partials(dst2, npad, steps):
    rows_per_tile = npad // _NS

    @functools.partial(
        pl.kernel,
        out_type=jax.ShapeDtypeStruct((_NC * npad,), jnp.float32),
        mesh=_mesh(),
        scratch_types=[
            pltpu.VMEM_SHARED((npad,), jnp.float32),
            pltpu.VMEM((steps, _CH), jnp.int32),
            pltpu.VMEM((_CH,), jnp.float32),
            pltpu.VMEM((rows_per_tile,), jnp.float32),
        ],
    )
    def k(dst_hbm, out_hbm, acc, dbuf, ones, stage):
        c = lax.axis_index("c")
        s = lax.axis_index("s")
        w = c * _NS + s

        @pl.loop(0, _CH, step=16)
        def _(i):
            ones[pl.ds(i, 16)] = jnp.ones((16,), jnp.float32)

        @pl.loop(0, rows_per_tile, step=16)
        def _(i):
            stage[pl.ds(i, 16)] = jnp.zeros((16,), jnp.float32)

        pltpu.sync_copy(stage, acc.at[pl.ds(s * rows_per_tile, rows_per_tile)])
        plsc.subcore_barrier()

        pltpu.sync_copy(dst_hbm.at[pl.ds(w * steps, steps)], dbuf)

        @pl.loop(0, steps)
        def _(g):
            pltpu.sync_copy(ones, acc.at[dbuf.at[g]], add=True)

        plsc.subcore_barrier()
        pltpu.sync_copy(acc.at[pl.ds(s * rows_per_tile, rows_per_tile)], stage)
        pltpu.sync_copy(
            stage, out_hbm.at[pl.ds(c * npad + s * rows_per_tile, rows_per_tile)])

    return k(dst2)


def _edge_partials(hws, src2, dst2, npad, steps):
    rows_per_tile = npad // _NS
    n_chunks = rows_per_tile // _CH
    ps = steps // _NPH

    @functools.partial(
        pl.kernel,
        out_type=jax.ShapeDtypeStruct((_NC, npad, _D), jnp.float32),
        mesh=_mesh(),
        scratch_types=[
            pltpu.VMEM_SHARED((npad, _D), jnp.float32),
            pltpu.VMEM((ps, _CH), jnp.int32),
            pltpu.VMEM((ps, _CH), jnp.int32),
            pltpu.VMEM((2, _CH, _D), jnp.float32),
            pltpu.SemaphoreType.DMA,
            pltpu.SemaphoreType.DMA,
            pltpu.SemaphoreType.DMA,
            pltpu.SemaphoreType.DMA,
        ],
    )
    def k(hws_hbm, src_hbm, dst_hbm, out_hbm, acc, sbuf, dbuf, rows,
          gsem0, gsem1, ssem0, ssem1):
        c = lax.axis_index("c")
        s = lax.axis_index("s")
        w = c * _NS + s
        gsems = (gsem0, gsem1)
        ssems = (ssem0, ssem1)

        @pl.loop(0, _CH)
        def _(i):
            @pl.loop(0, _D, step=16)
            def _(j):
                rows[0, i, pl.ds(j, 16)] = jnp.zeros((16,), jnp.float32)

        @pl.loop(0, n_chunks)
        def _(kk):
            pltpu.sync_copy(
                rows.at[0], acc.at[pl.ds(s * rows_per_tile + kk * _CH, _CH)])

        plsc.subcore_barrier()

        for ph in range(_NPH):
            base = w * steps + ph * ps
            pltpu.sync_copy(src_hbm.at[pl.ds(base, ps)], sbuf)
            pltpu.sync_copy(dst_hbm.at[pl.ds(base, ps)], dbuf)

            pltpu.async_copy(hws_hbm.at[sbuf.at[0]], rows.at[0], gsem0)

            @pl.loop(0, ps, step=2)
            def _(g):
                for b in (0, 1):
                    gg = g + b

                    @pl.when(gg + 1 < ps)
                    def _():
                        @pl.when(gg >= 1)
                        def _():
                            pltpu.make_async_copy(
                                rows.at[1 - b], acc.at[dbuf.at[gg - 1]],
                                ssems[1 - b]).wait()

                        pltpu.async_copy(
                            hws_hbm.at[sbuf.at[gg + 1]], rows.at[1 - b],
                            gsems[1 - b])

                    pltpu.make_async_copy(
                        hws_hbm.at[sbuf.at[gg]], rows.at[b], gsems[b]).wait()
                    pltpu.async_copy(
                        rows.at[b], acc.at[dbuf.at[gg]], ssems[b], add=True)

            pltpu.make_async_copy(
                rows.at[0], acc.at[dbuf.at[ps - 2]], ssem0).wait()
            pltpu.make_async_copy(
                rows.at[1], acc.at[dbuf.at[ps - 1]], ssem1).wait()

        plsc.subcore_barrier()

        @pl.loop(0, n_chunks)
        def _(kk):
            r0 = s * rows_per_tile + kk * _CH
            pltpu.sync_copy(acc.at[pl.ds(r0, _CH)], rows.at[0])
            pltpu.sync_copy(rows.at[0], out_hbm.at[c, pl.ds(r0, _CH)])

    return k(hws, src2, dst2)


_R = 2000


def _row_spec(r, d):
    return pl.BlockSpec((r, d), lambda i: (i, 0))


def _full_spec(shape):
    nd = len(shape)
    return pl.BlockSpec(shape, lambda i: (0,) * nd)


def _mm(x, w):
    n, d = x.shape

    def body(x_ref, w_ref, o_ref):
        o_ref[...] = jnp.dot(x_ref[...], w_ref[...],
                             preferred_element_type=jnp.float32)

    return pl.pallas_call(
        body,
        grid=(n // _R,),
        in_specs=[_row_spec(_R, d), _full_spec(w.shape)],
        out_specs=_row_spec(_R, d),
        out_shape=jax.ShapeDtypeStruct((n, d), jnp.float32),
    )(x, w)


def _scale(hw, d0, d1):
    n, d = hw.shape

    def body(hw_ref, d0_ref, d1_ref, o_ref):
        dinv = lax.rsqrt(d0_ref[...] + d1_ref[...] + 1.0)
        o_ref[...] = hw_ref[...] * dinv

    return pl.pallas_call(
        body,
        grid=(n // _R,),
        in_specs=[_row_spec(_R, d), _row_spec(_R, 1), _row_spec(_R, 1)],
        out_specs=_row_spec(_R, d),
        out_shape=jax.ShapeDtypeStruct((n, d), jnp.float32),
    )(hw, d0, d1)


def _norm_act(p0, p1, hws, d0, d1, b, g, be, pw):
    dinv = lax.rsqrt(d0 + d1 + 1.0)
    agg = dinv * (p0 + p1 + hws) + b
    mu = jnp.mean(agg, axis=-1, keepdims=True)
    var = jnp.mean((agg - mu) ** 2, axis=-1, keepdims=True)
    h = (agg - mu) * lax.rsqrt(var + 1e-5) * g + be
    return jnp.where(h > 0, h, h * pw)


def _part_spec(core, r, d):
    return pl.BlockSpec((1, r, d), lambda i, _c=core: (_c, i, 0))


def _mid(P, hws, d0, d1, b, g, be, pw, w2):
    n, d = hws.shape

    def body(p0_r, p1_r, hws_r, d0_r, d1_r, b_r, g_r, be_r, pw_r, w2_r, o_ref):
        h = _norm_act(p0_r[0], p1_r[0], hws_r[...], d0_r[...], d1_r[...],
                      b_r[...], g_r[...], be_r[...], pw_r[...])
        dinv = lax.rsqrt(d0_r[...] + d1_r[...] + 1.0)
        o_ref[...] = jnp.dot(h, w2_r[...],
                             preferred_element_type=jnp.float32) * dinv

    return pl.pallas_call(
        body,
        grid=(n // _R,),
        in_specs=[_part_spec(0, _R, d), _part_spec(1, _R, d), _row_spec(_R, d),
                  _row_spec(_R, 1), _row_spec(_R, 1),
                  _full_spec((1, d)), _full_spec((1, d)), _full_spec((1, d)),
                  _full_spec((1, d)), _full_spec((d, d))],
        out_specs=_row_spec(_R, d),
        out_shape=jax.ShapeDtypeStruct((n, d), jnp.float32),
    )(P, P, hws, d0, d1, b, g, be, pw, w2)


def _post(P, hws, d0, d1, b, g, be, pw):
    n, d = hws.shape

    def body(p0_r, p1_r, hws_r, d0_r, d1_r, b_r, g_r, be_r, pw_r, o_ref):
        o_ref[...] = _norm_act(p0_r[0], p1_r[0], hws_r[...], d0_r[...],
                               d1_r[...], b_r[...], g_r[...], be_r[...],
                               pw_r[...])

    return pl.pallas_call(
        body,
        grid=(n // _R,),
        in_specs=[_part_spec(0, _R, d), _part_spec(1, _R, d), _row_spec(_R, d),
                  _row_spec(_R, 1), _row_spec(_R, 1),
                  _full_spec((1, d)), _full_spec((1, d)), _full_spec((1, d)),
                  _full_spec((1, d))],
        out_specs=_row_spec(_R, d),
        out_shape=jax.ShapeDtypeStruct((n, d), jnp.float32),
    )(P, P, hws, d0, d1, b, g, be, pw)


def kernel(x, edge_index, W1, b1, g1, be1, pw, W2, b2, g2, be2):
    n, d = x.shape
    e = edge_index.shape[1]

    align = _NPH * 8
    steps = -(-e // (_NW * _CH * align)) * align
    epad = _NW * steps * _CH
    npad = -(-(n + 1) // (_NS * _CH)) * (_NS * _CH)

    src = edge_index[0]
    dst = edge_index[1]
    padn = epad - e
    pad_iota = jnp.arange(padn, dtype=jnp.int32)
    pad_dst = n + pad_iota % (npad - n)
    pad_src = pad_iota % n
    src2 = jnp.concatenate([src, pad_src]).reshape(-1, _CH)
    dst2 = jnp.concatenate([dst, pad_dst]).reshape(-1, _CH)

    degp = _deg_partials(dst2, npad, steps).reshape(_NC, npad)
    hw1 = _mm(x, W1)
    d0 = degp[0, :n, None]
    d1 = degp[1, :n, None]

    hws1 = _scale(hw1, d0, d1)
    P1 = _edge_partials(hws1, src2, dst2, npad, steps)
    b1r, g1r, be1r, pwr = (v.reshape(1, d) for v in (b1, g1, be1, pw))
    hws2 = _mid(P1, hws1, d0, d1, b1r, g1r, be1r, pwr, W2)

    P2 = _edge_partials(hws2, src2, dst2, npad, steps)
    b2r, g2r, be2r = (v.reshape(1, d) for v in (b2, g2, be2))
    return _post(P2, hws2, d0, d1, b2r, g2r, be2r, pwr)
